# baseline (speedup 1.0000x reference)
import jax
import jax.numpy as jnp
from jax.experimental import pallas as pl
from jax.experimental.pallas import tpu as pltpu

_N = 65536
_D = 64
_K = 512
_ITERS = 4
_T = 4096
_NT = _N // _T

_F32 = jnp.float32


def _dot(a, b, dims):
    return jax.lax.dot_general(a, b, (dims, ((), ())),
                               preferred_element_type=_F32)


def _body(xa_ref, out_ref, centers_ref, sums_ref, acc_ref):
    centers_ref[...] = xa_ref[0:_K, 0:_D]
    acc_ref[...] = jnp.zeros((1, 1), _F32)
    ones_d = jnp.ones((1, _D), _F32)

    for p in range(_ITERS + 1):
        final = p == _ITERS
        c = centers_ref[...]
        cm2 = -2.0 * c
        cn = _dot(ones_d, c * c, ((1,), (1,)))
        sums_ref[...] = jnp.zeros((_K, _D + 1), _F32)

        def tile(t, carry):
            xt = xa_ref[pl.ds(t * _T, _T), 0:_D]
            xat = xa_ref[pl.ds(t * _T, _T), :]
            q = _dot(xt, cm2, ((1,), (1,)))
            d2 = q + cn
            rowmin = jnp.min(d2, axis=1, keepdims=True)
            oh = (d2 == rowmin).astype(_F32)
            sums_ref[...] += _dot(oh, xat, ((0,), (0,)))
            if final:
                logits = -0.5 * q
                m = jnp.max(logits, axis=1, keepdims=True)
                lse = m + jnp.log(
                    jnp.sum(jnp.exp(logits - m), axis=1, keepdims=True))
                acc_ref[...] = acc_ref[...] + jnp.sum(lse)
            return carry

        jax.lax.fori_loop(0, _NT, tile, 0)

        if not final:
            cnt = sums_ref[:, _D:]
            newc = sums_ref[:, 0:_D] / jnp.maximum(cnt, 1.0)
            centers_ref[...] = jnp.where(cnt > 0.0, newc, c)

    cnt = sums_ref[:, _D:]
    occ = (cnt > 0.0).astype(_F32)
    ki = jax.lax.broadcasted_iota(jnp.int32, (_K, _K), 0)
    ji = jax.lax.broadcasted_iota(jnp.int32, (_K, _K), 1)
    tril = (ji < ki).astype(_F32)
    rank = _dot(tril, occ, ((1,), (0,)))
    rank_i = rank.astype(jnp.int32)
    oh_rank = (rank_i == ji).astype(_F32)
    c_rank = _dot(oh_rank, centers_ref[...], ((1,), (0,)))
    picked_sum = jnp.sum(sums_ref[:, 0:_D] * c_rank)
    out_ref[...] = (acc_ref[...] - picked_sum) / _N


def kernel(x):
    xa = jnp.concatenate([x, jnp.ones((_N, 1), _F32)], axis=1)
    out = pl.pallas_call(
        _body,
        out_shape=jax.ShapeDtypeStruct((1, 1), _F32),
        scratch_shapes=[
            pltpu.VMEM((_K, _D), _F32),
            pltpu.VMEM((_K, _D + 1), _F32),
            pltpu.VMEM((1, 1), _F32),
        ],
    )(xa)
    return out[0, 0]
